# Initial kernel scaffold; baseline (speedup 1.0000x reference)
import jax
import jax.numpy as jnp
from jax.experimental import pallas as pl


def kernel(x, edge_index, edge_attr, batch, gc0_W_rel, gc0_b_rel, gc0_W_root, gc1_W_rel, gc1_b_rel, gc1_W_root, gc2_W_rel, gc2_b_rel, gc2_W_root, asap_lin_W, asap_lin_b, le1_W, le1_b, le2_W, le3_W, le3_b, lin1_W, lin1_b, lin2_W, lin2_b):
    raise NotImplementedError("write your pallas kernel here")


# speedup vs baseline: 2.3781x; 2.3781x over previous
import functools

import jax
import jax.numpy as jnp
from jax.experimental import pallas as pl
from jax.experimental.pallas import tpu as pltpu

NEGV = -1e30
BD = 256

_PC = pl.pallas_call


def _dot(a, b, dims):
    return jax.lax.dot_general(
        a, b, (dims, ((), ())),
        preferred_element_type=jnp.float32,
    )


def _full(shape):
    ndim = len(shape)
    return pl.BlockSpec(shape, lambda i: (0,) * ndim)


def _rows(bshape):
    return pl.BlockSpec(bshape, lambda i: (i, 0))


def _gconv_body(b_ref, h_ref, hblk_ref, wrel_ref, brel_ref, wroot_ref,
                g_ref, bp_ref):
    i = pl.program_id(0)
    bblk = b_ref[...]
    agg = _dot(bblk, h_ref[...], ((1,), (0,)))
    g = _dot(agg, wrel_ref[...], ((1,), (1,)))
    g = g + brel_ref[...] + _dot(hblk_ref[...], wroot_ref[...], ((1,), (1,)))
    g_ref[...] = jnp.maximum(g, 0.0)
    n = bblk.shape[1]
    ii = jax.lax.broadcasted_iota(jnp.int32, (BD, n), 0) + i * BD
    jj = jax.lax.broadcasted_iota(jnp.int32, (BD, n), 1)
    eye = ii == jj
    dv = jnp.sum(jnp.where(eye, bblk, 0.0), axis=1, keepdims=True)
    bp_ref[...] = bblk + jnp.where(eye & (dv == 0.0), 1.0, 0.0)


def _gconv(B, h, wrel, brel, wroot):
    n, din = h.shape
    d = wrel.shape[0]
    return _PC(
        _gconv_body,
        grid=(n // BD,),
        in_specs=[_rows((BD, n)), _full((n, din)), _rows((BD, din)),
                  _full(wrel.shape), _full(brel.shape), _full(wroot.shape)],
        out_specs=[_rows((BD, d)), _rows((BD, n))],
        out_shape=[jax.ShapeDtypeStruct((n, d), jnp.float32),
                   jax.ShapeDtypeStruct((n, n), jnp.float32)],
    )(B, h, h, wrel, brel, wroot)


def _xq_body(bp_ref, g_ref, linw_ref, linb_ref, out_ref):
    n, d = g_ref.shape
    ch = 128

    def step(c, acc):
        mb = (bp_ref[:, pl.ds(c * ch, ch)] != 0.0).astype(jnp.float32)
        gb = g_ref[pl.ds(c * ch, ch), :]
        for s in range(ch):
            acc = jnp.maximum(acc, mb[:, s:s + 1] * gb[s:s + 1, :])
        return acc

    acc = jax.lax.fori_loop(0, n // ch, step,
                            jnp.zeros((BD, d), jnp.float32))
    out_ref[...] = _dot(acc, linw_ref[...], ((1,), (1,))) + linb_ref[...]


def _xq(Bp, g, linw, linb):
    n, d = g.shape
    return _PC(
        _xq_body,
        grid=(n // BD,),
        in_specs=[_rows((BD, n)), _full((n, d)),
                  _full(linw.shape), _full(linb.shape)],
        out_specs=_rows((BD, d)),
        out_shape=jax.ShapeDtypeStruct((n, d), jnp.float32),
    )(Bp, g, linw, linb)


def _attn_body(bp_ref, g_ref, xq2_ref, lew3_ref, leb3_ref,
               st_ref, xnew_ref, a_ref, b2_ref, c_ref):
    mt = bp_ref[...] != 0.0
    sc = _dot(xq2_ref[...], g_ref[...], ((1,), (1,)))
    sc = jnp.where(sc >= 0.0, sc, 0.2 * sc)
    sc = jnp.where(mt, sc, NEGV)
    mx = jnp.max(sc, axis=1, keepdims=True)
    e = jnp.exp(sc - mx)
    st = e / jnp.sum(e, axis=1, keepdims=True)
    st = jnp.where(mt, st, 0.0)
    st_ref[...] = st
    xnew = _dot(st, g_ref[...], ((1,), (0,)))
    xnew_ref[...] = xnew
    abc = _dot(xnew, lew3_ref[...], ((1,), (1,))) + leb3_ref[...]
    a_ref[...] = abc[:, 0:1]
    b2_ref[...] = abc[:, 1:2]
    c_ref[...] = abc[:, 2:3]


def _attn(Bp, g, xq2, lew3, leb3):
    n, d = g.shape
    return _PC(
        _attn_body,
        grid=(n // BD,),
        in_specs=[_rows((BD, n)), _full((n, d)), _rows((BD, d)),
                  _full(lew3.shape), _full(leb3.shape)],
        out_specs=[_rows((BD, n)), _rows((BD, d)), _rows((BD, 1)),
                   _rows((BD, 1)), _rows((BD, 1))],
        out_shape=[jax.ShapeDtypeStruct((n, n), jnp.float32),
                   jax.ShapeDtypeStruct((n, d), jnp.float32),
                   jax.ShapeDtypeStruct((n, 1), jnp.float32),
                   jax.ShapeDtypeStruct((n, 1), jnp.float32),
                   jax.ShapeDtypeStruct((n, 1), jnp.float32)],
    )(Bp, g, xq2, lew3, leb3)


def _fit_body(bp_ref, a_ref, b2_ref, c_ref, fit_ref):
    mf = (bp_ref[...] != 0.0).astype(jnp.float32)
    agg = _dot(mf, a_ref[...], ((1,), (0,)))
    deg = jnp.sum(mf, axis=1, keepdims=True)
    fit = agg - deg * b2_ref[...] + c_ref[...]
    fit_ref[...] = 1.0 / (1.0 + jnp.exp(-fit))


def _fit(Bp, a, b2, c):
    n = Bp.shape[0]
    return _PC(
        _fit_body,
        grid=(n // BD,),
        in_specs=[_rows((BD, n)), _full((n, 1)), _rows((BD, 1)),
                  _rows((BD, 1))],
        out_specs=_rows((BD, 1)),
        out_shape=jax.ShapeDtypeStruct((n, 1), jnp.float32),
    )(Bp, a, b2, c)


def _gather_body(perm_ref, st_ref, xn_ref, fit_ref, spt_ref, xo_ref):
    i = pl.program_id(0)
    for j in range(8):
        p = perm_ref[i * 8 + j]
        spt_ref[pl.ds(j, 1), :] = st_ref[pl.ds(p, 1), :]
        xo_ref[pl.ds(j, 1), :] = xn_ref[pl.ds(p, 1), :] * fit_ref[pl.ds(p, 1), :]


def _gather(perm, ST, xnew, fit):
    n, d = xnew.shape
    k = perm.shape[0]
    grid_spec = pltpu.PrefetchScalarGridSpec(
        num_scalar_prefetch=1,
        grid=(k // 8,),
        in_specs=[
            pl.BlockSpec((n, n), lambda i, perm: (0, 0)),
            pl.BlockSpec((n, d), lambda i, perm: (0, 0)),
            pl.BlockSpec((n, 1), lambda i, perm: (0, 0)),
        ],
        out_specs=[
            pl.BlockSpec((8, n), lambda i, perm: (i, 0)),
            pl.BlockSpec((8, d), lambda i, perm: (i, 0)),
        ],
    )
    return _PC(
        _gather_body,
        grid_spec=grid_spec,
        out_shape=[jax.ShapeDtypeStruct((k, n), jnp.float32),
                   jax.ShapeDtypeStruct((k, d), jnp.float32)],
    )(perm, ST, xnew, fit)


def _coarsen1_body(spt_ref, bp_ref, u_ref):
    u_ref[...] = _dot(spt_ref[...], bp_ref[...], ((1,), (0,)))


def _coarsen2_body(u_ref, spt_ref, b2_ref):
    i = pl.program_id(0)
    b2 = _dot(u_ref[...], spt_ref[...], ((1,), (1,)))
    k = b2.shape[1]
    ii = jax.lax.broadcasted_iota(jnp.int32, (b2.shape[0], k), 0) + i * b2.shape[0]
    jj = jax.lax.broadcasted_iota(jnp.int32, (b2.shape[0], k), 1)
    b2_ref[...] = jnp.where(ii == jj, 0.0, b2)


def _coarsen(Bp, SpT):
    n = Bp.shape[0]
    k = SpT.shape[0]
    bd2 = min(BD, k)
    u = _PC(
        _coarsen1_body,
        grid=(k // bd2,),
        in_specs=[pl.BlockSpec((bd2, n), lambda i: (i, 0)), _full((n, n))],
        out_specs=pl.BlockSpec((bd2, n), lambda i: (i, 0)),
        out_shape=jax.ShapeDtypeStruct((k, n), jnp.float32),
    )(SpT, Bp)
    return _PC(
        _coarsen2_body,
        grid=(k // bd2,),
        in_specs=[pl.BlockSpec((bd2, n), lambda i: (i, 0)), _full((k, n))],
        out_specs=pl.BlockSpec((bd2, k), lambda i: (i, 0)),
        out_shape=jax.ShapeDtypeStruct((k, k), jnp.float32),
    )(u, SpT)


def _readout_body(xo_ref, ro_ref):
    xo = xo_ref[...]
    d = xo.shape[1]
    ro_ref[:, :d] = jnp.mean(xo, axis=0, keepdims=True)
    ro_ref[:, d:] = jnp.max(xo, axis=0, keepdims=True)


def _readout(xo):
    k, d = xo.shape
    return _PC(
        _readout_body,
        grid=(1,),
        in_specs=[_full((k, d))],
        out_specs=_full((1, 2 * d)),
        out_shape=jax.ShapeDtypeStruct((1, 2 * d), jnp.float32),
    )(xo)


def _final_body(r0_ref, r1_ref, r2_ref, w1_ref, b1_ref, w2_ref, b2_ref,
                out_ref):
    ro = r0_ref[...] + r1_ref[...] + r2_ref[...]
    z = _dot(ro, w1_ref[...], ((1,), (1,))) + b1_ref[...]
    z = jnp.maximum(z, 0.0)
    out_ref[...] = _dot(z, w2_ref[...], ((1,), (1,))) + b2_ref[...]


def _final(r0, r1, r2, w1, b1, w2, b2):
    dout = w2.shape[0]
    return _PC(
        _final_body,
        grid=(1,),
        in_specs=[_full(r0.shape), _full(r1.shape), _full(r2.shape),
                  _full(w1.shape), _full(b1.shape), _full(w2.shape),
                  _full(b2.shape)],
        out_specs=_full((1, dout)),
        out_shape=jax.ShapeDtypeStruct((1, dout), jnp.float32),
    )(r0, r1, r2, w1, b1, w2, b2)


def kernel(x, edge_index, edge_attr, batch, gc0_W_rel, gc0_b_rel, gc0_W_root,
           gc1_W_rel, gc1_b_rel, gc1_W_root, gc2_W_rel, gc2_b_rel, gc2_W_root,
           asap_lin_W, asap_lin_b, le1_W, le1_b, le2_W, le3_W, le3_b,
           lin1_W, lin1_b, lin2_W, lin2_b):
    n0 = x.shape[0]
    B = jnp.zeros((n0, n0), jnp.float32).at[
        edge_index[1], edge_index[0]].add(edge_attr)

    brel = [gc0_b_rel[None, :], gc1_b_rel[None, :], gc2_b_rel[None, :]]
    wrel = [gc0_W_rel, gc1_W_rel, gc2_W_rel]
    wroot = [gc0_W_root, gc1_W_root, gc2_W_root]
    linb = asap_lin_b[None, :]
    lew3 = jnp.concatenate([le1_W, le2_W, le3_W], axis=0)
    leb3 = jnp.concatenate(
        [le1_b, jnp.zeros_like(le1_b), le3_b])[None, :]

    h = x
    ro = []
    for r in range(3):
        g, Bp = _gconv(B, h, wrel[r], brel[r], wroot[r])
        xq2 = _xq(Bp, g, asap_lin_W, linb)
        ST, xnew, a, b2, c = _attn(Bp, g, xq2, lew3, leb3)
        fit = _fit(Bp, a, b2, c)
        k = h.shape[0] // 2
        _, perm = jax.lax.top_k(fit[:, 0], k)
        SpT, x_out = _gather(perm, ST, xnew, fit)
        B = _coarsen(Bp, SpT)
        ro.append(_readout(x_out))
        h = x_out

    return _final(ro[0], ro[1], ro[2], lin1_W, lin1_b[None, :],
                  lin2_W, lin2_b[None, :])


# speedup vs baseline: 2.5272x; 1.0627x over previous
import functools

import jax
import jax.numpy as jnp
from jax.experimental import pallas as pl
from jax.experimental.pallas import tpu as pltpu

NEGV = -1e30
BD = 256

_PC = pl.pallas_call


def _dot(a, b, dims):
    return jax.lax.dot_general(
        a, b, (dims, ((), ())),
        preferred_element_type=jnp.float32,
    )


def _full(shape):
    ndim = len(shape)
    return pl.BlockSpec(shape, lambda i: (0,) * ndim)


def _rows(bshape):
    return pl.BlockSpec(bshape, lambda i: (i, 0))


def _gconv_body(b_ref, h_ref, hblk_ref, wrel_ref, brel_ref, wroot_ref,
                g_ref, bp_ref):
    i = pl.program_id(0)
    bblk = b_ref[...]
    agg = _dot(bblk, h_ref[...], ((1,), (0,)))
    g = _dot(agg, wrel_ref[...], ((1,), (1,)))
    g = g + brel_ref[...] + _dot(hblk_ref[...], wroot_ref[...], ((1,), (1,)))
    g_ref[...] = jnp.maximum(g, 0.0)
    n = bblk.shape[1]
    ii = jax.lax.broadcasted_iota(jnp.int32, (BD, n), 0) + i * BD
    jj = jax.lax.broadcasted_iota(jnp.int32, (BD, n), 1)
    eye = ii == jj
    dv = jnp.sum(jnp.where(eye, bblk, 0.0), axis=1, keepdims=True)
    bp_ref[...] = bblk + jnp.where(eye & (dv == 0.0), 1.0, 0.0)


def _gconv(B, h, wrel, brel, wroot):
    n, din = h.shape
    d = wrel.shape[0]
    return _PC(
        _gconv_body,
        grid=(n // BD,),
        in_specs=[_rows((BD, n)), _full((n, din)), _rows((BD, din)),
                  _full(wrel.shape), _full(brel.shape), _full(wroot.shape)],
        out_specs=[_rows((BD, d)), _rows((BD, n))],
        out_shape=[jax.ShapeDtypeStruct((n, d), jnp.float32),
                   jax.ShapeDtypeStruct((n, n), jnp.float32)],
    )(B, h, h, wrel, brel, wroot)


def _xq_body(bp_ref, g_ref, linw_ref, linb_ref, out_ref):
    n, d = g_ref.shape
    ch = 128

    def step(c, acc):
        mb = (bp_ref[:, pl.ds(c * ch, ch)] != 0.0).astype(jnp.float32)
        gb = g_ref[pl.ds(c * ch, ch), :]
        for s in range(ch):
            acc = jnp.maximum(acc, mb[:, s:s + 1] * gb[s:s + 1, :])
        return acc

    acc = jax.lax.fori_loop(0, n // ch, step,
                            jnp.zeros((BD, d), jnp.float32))
    out_ref[...] = _dot(acc, linw_ref[...], ((1,), (1,))) + linb_ref[...]


def _xq(Bp, g, linw, linb):
    n, d = g.shape
    return _PC(
        _xq_body,
        grid=(n // BD,),
        in_specs=[_rows((BD, n)), _full((n, d)),
                  _full(linw.shape), _full(linb.shape)],
        out_specs=_rows((BD, d)),
        out_shape=jax.ShapeDtypeStruct((n, d), jnp.float32),
    )(Bp, g, linw, linb)


def _attn_body(bp_ref, g_ref, xq2_ref, lew3_ref, leb3_ref,
               st_ref, xnew_ref, a_ref, b2_ref, c_ref):
    mt = bp_ref[...] != 0.0
    sc = _dot(xq2_ref[...], g_ref[...], ((1,), (1,)))
    sc = jnp.where(sc >= 0.0, sc, 0.2 * sc)
    sc = jnp.where(mt, sc, NEGV)
    mx = jnp.max(sc, axis=1, keepdims=True)
    e = jnp.exp(sc - mx)
    st = e / jnp.sum(e, axis=1, keepdims=True)
    st = jnp.where(mt, st, 0.0)
    st_ref[...] = st
    xnew = _dot(st, g_ref[...], ((1,), (0,)))
    xnew_ref[...] = xnew
    abc = _dot(xnew, lew3_ref[...], ((1,), (1,))) + leb3_ref[...]
    a_ref[...] = abc[:, 0:1]
    b2_ref[...] = abc[:, 1:2]
    c_ref[...] = abc[:, 2:3]


def _attn(Bp, g, xq2, lew3, leb3):
    n, d = g.shape
    return _PC(
        _attn_body,
        grid=(n // BD,),
        in_specs=[_rows((BD, n)), _full((n, d)), _rows((BD, d)),
                  _full(lew3.shape), _full(leb3.shape)],
        out_specs=[_rows((BD, n)), _rows((BD, d)), _rows((BD, 1)),
                   _rows((BD, 1)), _rows((BD, 1))],
        out_shape=[jax.ShapeDtypeStruct((n, n), jnp.float32),
                   jax.ShapeDtypeStruct((n, d), jnp.float32),
                   jax.ShapeDtypeStruct((n, 1), jnp.float32),
                   jax.ShapeDtypeStruct((n, 1), jnp.float32),
                   jax.ShapeDtypeStruct((n, 1), jnp.float32)],
    )(Bp, g, xq2, lew3, leb3)


def _fit_body(bp_ref, a_ref, b2_ref, c_ref, fit_ref):
    mf = (bp_ref[...] != 0.0).astype(jnp.float32)
    agg = _dot(mf, a_ref[...], ((1,), (0,)))
    deg = jnp.sum(mf, axis=1, keepdims=True)
    fit = agg - deg * b2_ref[...] + c_ref[...]
    fit_ref[...] = 1.0 / (1.0 + jnp.exp(-fit))


def _fit(Bp, a, b2, c):
    n = Bp.shape[0]
    return _PC(
        _fit_body,
        grid=(n // BD,),
        in_specs=[_rows((BD, n)), _full((n, 1)), _rows((BD, 1)),
                  _rows((BD, 1))],
        out_specs=_rows((BD, 1)),
        out_shape=jax.ShapeDtypeStruct((n, 1), jnp.float32),
    )(Bp, a, b2, c)


RB = 32


def _gather_body(perm_ref, st_ref, xn_ref, fit_ref, spt_ref, xo_ref):
    i = pl.program_id(0)
    for j in range(RB):
        p = perm_ref[i * RB + j]
        spt_ref[pl.ds(j, 1), :] = st_ref[pl.ds(p, 1), :]
        xo_ref[pl.ds(j, 1), :] = xn_ref[pl.ds(p, 1), :] * fit_ref[pl.ds(p, 1), :]


def _gather(perm, ST, xnew, fit):
    n, d = xnew.shape
    k = perm.shape[0]
    grid_spec = pltpu.PrefetchScalarGridSpec(
        num_scalar_prefetch=1,
        grid=(k // RB,),
        in_specs=[
            pl.BlockSpec((n, n), lambda i, perm: (0, 0)),
            pl.BlockSpec((n, d), lambda i, perm: (0, 0)),
            pl.BlockSpec((n, 1), lambda i, perm: (0, 0)),
        ],
        out_specs=[
            pl.BlockSpec((RB, n), lambda i, perm: (i, 0)),
            pl.BlockSpec((RB, d), lambda i, perm: (i, 0)),
        ],
    )
    return _PC(
        _gather_body,
        grid_spec=grid_spec,
        out_shape=[jax.ShapeDtypeStruct((k, n), jnp.float32),
                   jax.ShapeDtypeStruct((k, d), jnp.float32)],
    )(perm, ST, xnew, fit)


def _coarsen1_body(spt_ref, bp_ref, u_ref):
    u_ref[...] = _dot(spt_ref[...], bp_ref[...], ((1,), (0,)))


def _coarsen2_body(u_ref, spt_ref, b2_ref):
    i = pl.program_id(0)
    b2 = _dot(u_ref[...], spt_ref[...], ((1,), (1,)))
    k = b2.shape[1]
    ii = jax.lax.broadcasted_iota(jnp.int32, (b2.shape[0], k), 0) + i * b2.shape[0]
    jj = jax.lax.broadcasted_iota(jnp.int32, (b2.shape[0], k), 1)
    b2_ref[...] = jnp.where(ii == jj, 0.0, b2)


def _coarsen(Bp, SpT):
    n = Bp.shape[0]
    k = SpT.shape[0]
    bd2 = min(BD, k)
    u = _PC(
        _coarsen1_body,
        grid=(k // bd2,),
        in_specs=[pl.BlockSpec((bd2, n), lambda i: (i, 0)), _full((n, n))],
        out_specs=pl.BlockSpec((bd2, n), lambda i: (i, 0)),
        out_shape=jax.ShapeDtypeStruct((k, n), jnp.float32),
    )(SpT, Bp)
    return _PC(
        _coarsen2_body,
        grid=(k // bd2,),
        in_specs=[pl.BlockSpec((bd2, n), lambda i: (i, 0)), _full((k, n))],
        out_specs=pl.BlockSpec((bd2, k), lambda i: (i, 0)),
        out_shape=jax.ShapeDtypeStruct((k, k), jnp.float32),
    )(u, SpT)


def _readout_body(xo_ref, ro_ref):
    xo = xo_ref[...]
    d = xo.shape[1]
    ro_ref[:, :d] = jnp.mean(xo, axis=0, keepdims=True)
    ro_ref[:, d:] = jnp.max(xo, axis=0, keepdims=True)


def _readout(xo):
    k, d = xo.shape
    return _PC(
        _readout_body,
        grid=(1,),
        in_specs=[_full((k, d))],
        out_specs=_full((1, 2 * d)),
        out_shape=jax.ShapeDtypeStruct((1, 2 * d), jnp.float32),
    )(xo)


def _final_body(r0_ref, r1_ref, r2_ref, w1_ref, b1_ref, w2_ref, b2_ref,
                out_ref):
    ro = r0_ref[...] + r1_ref[...] + r2_ref[...]
    z = _dot(ro, w1_ref[...], ((1,), (1,))) + b1_ref[...]
    z = jnp.maximum(z, 0.0)
    out_ref[...] = _dot(z, w2_ref[...], ((1,), (1,))) + b2_ref[...]


def _final(r0, r1, r2, w1, b1, w2, b2):
    dout = w2.shape[0]
    return _PC(
        _final_body,
        grid=(1,),
        in_specs=[_full(r0.shape), _full(r1.shape), _full(r2.shape),
                  _full(w1.shape), _full(b1.shape), _full(w2.shape),
                  _full(b2.shape)],
        out_specs=_full((1, dout)),
        out_shape=jax.ShapeDtypeStruct((1, dout), jnp.float32),
    )(r0, r1, r2, w1, b1, w2, b2)


def kernel(x, edge_index, edge_attr, batch, gc0_W_rel, gc0_b_rel, gc0_W_root,
           gc1_W_rel, gc1_b_rel, gc1_W_root, gc2_W_rel, gc2_b_rel, gc2_W_root,
           asap_lin_W, asap_lin_b, le1_W, le1_b, le2_W, le3_W, le3_b,
           lin1_W, lin1_b, lin2_W, lin2_b):
    n0 = x.shape[0]
    B = jnp.zeros((n0, n0), jnp.float32).at[
        edge_index[1], edge_index[0]].add(edge_attr)

    brel = [gc0_b_rel[None, :], gc1_b_rel[None, :], gc2_b_rel[None, :]]
    wrel = [gc0_W_rel, gc1_W_rel, gc2_W_rel]
    wroot = [gc0_W_root, gc1_W_root, gc2_W_root]
    linb = asap_lin_b[None, :]
    lew3 = jnp.concatenate([le1_W, le2_W, le3_W], axis=0)
    leb3 = jnp.concatenate(
        [le1_b, jnp.zeros_like(le1_b), le3_b])[None, :]

    h = x
    ro = []
    for r in range(3):
        g, Bp = _gconv(B, h, wrel[r], brel[r], wroot[r])
        xq2 = _xq(Bp, g, asap_lin_W, linb)
        ST, xnew, a, b2, c = _attn(Bp, g, xq2, lew3, leb3)
        fit = _fit(Bp, a, b2, c)
        k = h.shape[0] // 2
        _, perm = jax.lax.top_k(fit[:, 0], k)
        SpT, x_out = _gather(perm, ST, xnew, fit)
        B = _coarsen(Bp, SpT)
        ro.append(_readout(x_out))
        h = x_out

    return _final(ro[0], ro[1], ro[2], lin1_W, lin1_b[None, :],
                  lin2_W, lin2_b[None, :])
